# baseline (speedup 1.0000x reference)
import functools
import math

import jax
import jax.numpy as jnp
from jax import lax
from jax.experimental import pallas as pl
from jax.experimental.pallas import tpu as pltpu
from jax.experimental.pallas import tpu_sc as plsc

VOCAB_D = 64
SCALE = math.sqrt(VOCAB_D)

NC = 2
NS = 16
NW = NC * NS
IDXW = 128
SUB = 8
CHUNK = IDXW * SUB


@functools.partial(jax.jit, static_argnames=())
def _embed(x_flat, table):
    B = x_flat.shape[0]
    D = table.shape[1]
    b_per_w = B // NW
    n_chunks = b_per_w // CHUNK
    idx2 = x_flat.reshape(B // IDXW, IDXW)

    mesh = plsc.VectorSubcoreMesh(core_axis_name="c", subcore_axis_name="s")

    @functools.partial(
        pl.kernel,
        mesh=mesh,
        out_type=jax.ShapeDtypeStruct((B, D), jnp.float32),
        scratch_types=[
            pltpu.VMEM((SUB, IDXW), jnp.int32),
            pltpu.VMEM((CHUNK, D), jnp.float32),
            pltpu.SemaphoreType.DMA,
        ],
        compiler_params=pltpu.CompilerParams(use_tc_tiling_on_sc=False),
    )
    def body(table_hbm, idx_hbm, out_hbm, idx_v, rows_v, sem):
        wid = lax.axis_index("s") * NC + lax.axis_index("c")
        row_base = wid * b_per_w
        idx_base = wid * (b_per_w // IDXW)

        def chunk_body(g, carry):
            off = row_base + g * CHUNK
            ioff = idx_base + g * SUB
            pltpu.sync_copy(idx_hbm.at[pl.ds(ioff, SUB)], idx_v)
            copies = [
                pltpu.async_copy(
                    table_hbm.at[idx_v.at[j]],
                    rows_v.at[pl.ds(j * IDXW, IDXW)],
                    sem,
                )
                for j in range(SUB)
            ]
            for c in copies:
                c.wait()

            def scale_body(r, c2):
                for k in range(D // 16):
                    sl = pl.ds(k * 16, 16)
                    rows_v[r, sl] = rows_v[r, sl] * SCALE
                return c2

            lax.fori_loop(0, CHUNK, scale_body, 0, unroll=4)
            pltpu.sync_copy(rows_v, out_hbm.at[pl.ds(off, CHUNK)])
            return carry

        lax.fori_loop(0, n_chunks, chunk_body, 0)

    return body(table, idx2)


def kernel(x, table):
    x_flat = x.reshape(-1).astype(jnp.int32)
    out = _embed(x_flat, table)
    return out.reshape(x.shape + (table.shape[1],))
